# baseline (device time: 152493 ns/iter reference)
import functools

import jax
import jax.numpy as jnp
from jax import lax
from jax.experimental import pallas as pl
from jax.experimental.pallas import tpu as pltpu

N_DEV = 8
_MASKS = (1, 3, 4)


def kernel(A, B):
    m, k = A.shape
    _, n = B.shape

    def body(a_ref, b_ref, out_ref, comm_ref, send_sems, recv_sems):
        my = lax.axis_index("i")
        partners = [jnp.bitwise_xor(my, mask) for mask in _MASKS]

        barrier_sem = pltpu.get_barrier_semaphore()
        for p in partners:
            pl.semaphore_signal(
                barrier_sem, inc=1,
                device_id=(p,), device_id_type=pl.DeviceIdType.MESH,
            )
        pl.semaphore_wait(barrier_sem, len(partners))

        out_ref[...] = jnp.dot(
            a_ref[...], b_ref[...], preferred_element_type=jnp.float32
        )

        for s, p in enumerate(partners):
            rdma = pltpu.make_async_remote_copy(
                src_ref=out_ref,
                dst_ref=comm_ref.at[s],
                send_sem=send_sems.at[s],
                recv_sem=recv_sems.at[s],
                device_id=(p,),
                device_id_type=pl.DeviceIdType.MESH,
            )
            rdma.start()
            rdma.wait()
            out_ref[...] = out_ref[...] + comm_ref[s]

        @functools.partial(
            pl.run_scoped, second_barrier=pltpu.SemaphoreType.REGULAR
        )
        def _(second_barrier):
            for p in partners:
                pl.semaphore_signal(
                    second_barrier, inc=1,
                    device_id=(p,), device_id_type=pl.DeviceIdType.MESH,
                )
            pl.semaphore_wait(second_barrier, len(partners))

    return pl.pallas_call(
        body,
        out_shape=jax.ShapeDtypeStruct((m, n), jnp.float32),
        in_specs=[
            pl.BlockSpec(memory_space=pltpu.VMEM),
            pl.BlockSpec(memory_space=pltpu.VMEM),
        ],
        out_specs=pl.BlockSpec(memory_space=pltpu.VMEM),
        scratch_shapes=[
            pltpu.VMEM((len(_MASKS), m, n), jnp.float32),
            pltpu.SemaphoreType.DMA((len(_MASKS),)),
            pltpu.SemaphoreType.DMA((len(_MASKS),)),
        ],
        compiler_params=pltpu.CompilerParams(collective_id=0),
    )(A, B)


# device time: 51075 ns/iter; 2.9857x vs baseline; 2.9857x over previous
import functools

import jax
import jax.numpy as jnp
from jax import lax
from jax.experimental import pallas as pl
from jax.experimental.pallas import tpu as pltpu

N_DEV = 8
_MASKS = (1, 3, 4)
_PART_ROWS = (384, 320, 320)
_PART_OFFS = (0, 384, 704)


def kernel(A, B):
    m, k = A.shape
    _, n = B.shape

    comm_offs = [[0] * 3 for _ in range(3)]
    acc = 0
    for p in range(3):
        for s in range(3):
            comm_offs[p][s] = acc
            acc += _PART_ROWS[p] >> (s + 1)
    comm_rows = acc

    def body(a_ref, b_ref, out_ref, comm_ref, rs_send, rs_recv,
             ag_send, ag_recv):
        my = lax.axis_index("i")
        bit = [
            (my ^ (my >> 1)) & 1,
            (my >> 1) & 1,
            (my >> 2) & 1,
        ]
        partner = [my ^ mask for mask in _MASKS]

        barrier_sem = pltpu.get_barrier_semaphore()
        for pt in partner:
            pl.semaphore_signal(
                barrier_sem, inc=1,
                device_id=(pt,), device_id_type=pl.DeviceIdType.MESH,
            )
        pl.semaphore_wait(barrier_sem, len(partner))

        out_ref[...] = jnp.dot(
            a_ref[...], b_ref[...], preferred_element_type=jnp.float32
        )

        offs = [jnp.int32(_PART_OFFS[p]) for p in range(3)]

        for s in range(3):
            rdmas = []
            for p in range(3):
                d = (p + s) % 3
                half = _PART_ROWS[p] >> (s + 1)
                b = bit[d]
                send_off = offs[p] + (1 - b) * half
                rdma = pltpu.make_async_remote_copy(
                    src_ref=out_ref.at[pl.ds(send_off, half)],
                    dst_ref=comm_ref.at[pl.ds(comm_offs[p][s], half)],
                    send_sem=rs_send.at[p, s],
                    recv_sem=rs_recv.at[p, s],
                    device_id=(partner[d],),
                    device_id_type=pl.DeviceIdType.MESH,
                )
                rdma.start()
                rdmas.append(rdma)
                offs[p] = offs[p] + b * half
            for p in range(3):
                rdmas[p].wait()
            for p in range(3):
                half = _PART_ROWS[p] >> (s + 1)
                out_ref[pl.ds(offs[p], half), :] = (
                    out_ref[pl.ds(offs[p], half), :]
                    + comm_ref[pl.ds(comm_offs[p][s], half), :]
                )

        for t in range(3):
            rdmas = []
            for p in range(3):
                d = (p + 2 - t) % 3
                sz = (_PART_ROWS[p] >> 3) << t
                b = bit[d]
                rdma = pltpu.make_async_remote_copy(
                    src_ref=out_ref.at[pl.ds(offs[p], sz)],
                    dst_ref=out_ref.at[pl.ds(offs[p], sz)],
                    send_sem=ag_send.at[p, t],
                    recv_sem=ag_recv.at[p, t],
                    device_id=(partner[d],),
                    device_id_type=pl.DeviceIdType.MESH,
                )
                rdma.start()
                rdmas.append(rdma)
                offs[p] = offs[p] - b * sz
            for p in range(3):
                rdmas[p].wait()

        @functools.partial(
            pl.run_scoped, second_barrier=pltpu.SemaphoreType.REGULAR
        )
        def _(second_barrier):
            for pt in partner:
                pl.semaphore_signal(
                    second_barrier, inc=1,
                    device_id=(pt,), device_id_type=pl.DeviceIdType.MESH,
                )
            pl.semaphore_wait(second_barrier, len(partner))

    return pl.pallas_call(
        body,
        out_shape=jax.ShapeDtypeStruct((m, n), jnp.float32),
        in_specs=[
            pl.BlockSpec(memory_space=pltpu.VMEM),
            pl.BlockSpec(memory_space=pltpu.VMEM),
        ],
        out_specs=pl.BlockSpec(memory_space=pltpu.VMEM),
        scratch_shapes=[
            pltpu.VMEM((comm_rows, n), jnp.float32),
            pltpu.SemaphoreType.DMA((3, 3)),
            pltpu.SemaphoreType.DMA((3, 3)),
            pltpu.SemaphoreType.DMA((3, 3)),
            pltpu.SemaphoreType.DMA((3, 3)),
        ],
        compiler_params=pltpu.CompilerParams(collective_id=0),
    )(A, B)


# device time: 50041 ns/iter; 3.0474x vs baseline; 1.0207x over previous
import functools

import jax
import jax.numpy as jnp
from jax import lax
from jax.experimental import pallas as pl
from jax.experimental.pallas import tpu as pltpu

N_DEV = 8
_MASKS = (1, 3, 4)
_PART_ROWS = (384, 320, 320)
_PART_OFFS = (0, 384, 704)


def kernel(A, B):
    m, k = A.shape
    _, n = B.shape

    comm_offs = [[0] * 3 for _ in range(3)]
    acc = 0
    for p in range(3):
        for s in range(3):
            comm_offs[p][s] = acc
            acc += _PART_ROWS[p] >> (s + 1)
    comm_rows = acc

    def body(a_ref, b_ref, out_ref, comm_ref, rs_send, rs_recv,
             ag_send, ag_recv):
        my = lax.axis_index("i")
        bit = [
            (my ^ (my >> 1)) & 1,
            (my >> 1) & 1,
            (my >> 2) & 1,
        ]
        partner = [my ^ mask for mask in _MASKS]

        barrier_sem = pltpu.get_barrier_semaphore()
        for pt in partner:
            pl.semaphore_signal(
                barrier_sem, inc=1,
                device_id=(pt,), device_id_type=pl.DeviceIdType.MESH,
            )
        pl.semaphore_wait(barrier_sem, len(partner))

        offs = [jnp.int32(_PART_OFFS[p]) for p in range(3)]
        all_rdmas = []

        def start_rs(p, s):
            d = (p + s) % 3
            half = _PART_ROWS[p] >> (s + 1)
            b = bit[d]
            send_off = offs[p] + (1 - b) * half
            rdma = pltpu.make_async_remote_copy(
                src_ref=out_ref.at[pl.ds(send_off, half)],
                dst_ref=comm_ref.at[pl.ds(comm_offs[p][s], half)],
                send_sem=rs_send.at[p, s],
                recv_sem=rs_recv.at[p, s],
                device_id=(partner[d],),
                device_id_type=pl.DeviceIdType.MESH,
            )
            rdma.start()
            all_rdmas.append(rdma)
            offs[p] = offs[p] + b * half
            return rdma

        def start_ag(p, t):
            d = (p + 2 - t) % 3
            sz = (_PART_ROWS[p] >> 3) << t
            b = bit[d]
            rdma = pltpu.make_async_remote_copy(
                src_ref=out_ref.at[pl.ds(offs[p], sz)],
                dst_ref=out_ref.at[pl.ds(offs[p], sz)],
                send_sem=ag_send.at[p, t],
                recv_sem=ag_recv.at[p, t],
                device_id=(partner[d],),
                device_id_type=pl.DeviceIdType.MESH,
            )
            rdma.start()
            all_rdmas.append(rdma)
            offs[p] = offs[p] - b * sz
            return rdma

        def rs_add(p, s):
            half = _PART_ROWS[p] >> (s + 1)
            out_ref[pl.ds(offs[p], half), :] = (
                out_ref[pl.ds(offs[p], half), :]
                + comm_ref[pl.ds(comm_offs[p][s], half), :]
            )

        rs_rdmas = {}
        for p in range(3):
            r0, nr = _PART_OFFS[p], _PART_ROWS[p]
            out_ref[pl.ds(r0, nr), :] = jnp.dot(
                a_ref[pl.ds(r0, nr), :], b_ref[...],
                preferred_element_type=jnp.float32,
            )
            rs_rdmas[p] = start_rs(p, 0)

        ag_rdmas = {}
        for s in range(3):
            for p in (1, 2, 0):
                rs_rdmas[p].wait_recv()
                rs_add(p, s)
                if s < 2:
                    rs_rdmas[p] = start_rs(p, s + 1)
                else:
                    ag_rdmas[p] = start_ag(p, 0)

        for t in range(3):
            for p in (1, 2, 0):
                ag_rdmas[p].wait_recv()
                if t < 2:
                    ag_rdmas[p] = start_ag(p, t + 1)

        for rdma in all_rdmas:
            rdma.wait_send()

        @functools.partial(
            pl.run_scoped, second_barrier=pltpu.SemaphoreType.REGULAR
        )
        def _(second_barrier):
            for pt in partner:
                pl.semaphore_signal(
                    second_barrier, inc=1,
                    device_id=(pt,), device_id_type=pl.DeviceIdType.MESH,
                )
            pl.semaphore_wait(second_barrier, len(partner))

    return pl.pallas_call(
        body,
        out_shape=jax.ShapeDtypeStruct((m, n), jnp.float32),
        in_specs=[
            pl.BlockSpec(memory_space=pltpu.VMEM),
            pl.BlockSpec(memory_space=pltpu.VMEM),
        ],
        out_specs=pl.BlockSpec(memory_space=pltpu.VMEM),
        scratch_shapes=[
            pltpu.VMEM((comm_rows, n), jnp.float32),
            pltpu.SemaphoreType.DMA((3, 3)),
            pltpu.SemaphoreType.DMA((3, 3)),
            pltpu.SemaphoreType.DMA((3, 3)),
            pltpu.SemaphoreType.DMA((3, 3)),
        ],
        compiler_params=pltpu.CompilerParams(collective_id=0),
    )(A, B)


# device time: 41137 ns/iter; 3.7070x vs baseline; 1.2164x over previous
import functools

import jax
import jax.numpy as jnp
from jax import lax
from jax.experimental import pallas as pl
from jax.experimental.pallas import tpu as pltpu

N_DEV = 8
_MASKS = (1, 3, 4)
_CHUNK_ROWS = (128, 64, 64, 128, 128, 128, 128, 128, 128)
_CHUNK_ROT = (0, 1, 2, 0, 1, 2, 0, 1, 2)
_CHUNK_OFFS = (0, 384, 704, 128, 448, 768, 256, 576, 896)
_N_CHUNKS = 9


def kernel(A, B):
    m, k = A.shape
    _, n = B.shape

    comm_offs = [[0] * 3 for _ in range(_N_CHUNKS)]
    acc = 0
    for c in range(_N_CHUNKS):
        for s in range(3):
            comm_offs[c][s] = acc
            acc += _CHUNK_ROWS[c] >> (s + 1)
    comm_rows = acc

    def body(a_ref, b_ref, out_ref, comm_ref, rs_send, rs_recv,
             ag_send, ag_recv):
        my = lax.axis_index("i")
        bit = [
            (my ^ (my >> 1)) & 1,
            (my >> 1) & 1,
            (my >> 2) & 1,
        ]
        partner = [my ^ mask for mask in _MASKS]

        barrier_sem = pltpu.get_barrier_semaphore()
        for pt in partner:
            pl.semaphore_signal(
                barrier_sem, inc=1,
                device_id=(pt,), device_id_type=pl.DeviceIdType.MESH,
            )
        pl.semaphore_wait(barrier_sem, len(partner))

        offs = [jnp.int32(_CHUNK_OFFS[c]) for c in range(_N_CHUNKS)]
        cur = [None] * _N_CHUNKS
        all_rdmas = []

        def start_rs(c, s):
            d = (_CHUNK_ROT[c] + s) % 3
            half = _CHUNK_ROWS[c] >> (s + 1)
            b = bit[d]
            send_off = offs[c] + (1 - b) * half
            rdma = pltpu.make_async_remote_copy(
                src_ref=out_ref.at[pl.ds(send_off, half)],
                dst_ref=comm_ref.at[pl.ds(comm_offs[c][s], half)],
                send_sem=rs_send.at[c, s],
                recv_sem=rs_recv.at[c, s],
                device_id=(partner[d],),
                device_id_type=pl.DeviceIdType.MESH,
            )
            rdma.start()
            all_rdmas.append(rdma)
            offs[c] = offs[c] + b * half
            cur[c] = rdma

        def start_ag(c, t):
            d = (_CHUNK_ROT[c] + 2 - t) % 3
            sz = (_CHUNK_ROWS[c] >> 3) << t
            b = bit[d]
            rdma = pltpu.make_async_remote_copy(
                src_ref=out_ref.at[pl.ds(offs[c], sz)],
                dst_ref=out_ref.at[pl.ds(offs[c], sz)],
                send_sem=ag_send.at[c, t],
                recv_sem=ag_recv.at[c, t],
                device_id=(partner[d],),
                device_id_type=pl.DeviceIdType.MESH,
            )
            rdma.start()
            all_rdmas.append(rdma)
            offs[c] = offs[c] - b * sz
            cur[c] = rdma

        def rs_add(c, s):
            half = _CHUNK_ROWS[c] >> (s + 1)
            out_ref[pl.ds(offs[c], half), :] = (
                out_ref[pl.ds(offs[c], half), :]
                + comm_ref[pl.ds(comm_offs[c][s], half), :]
            )

        def do_step(c, g):
            if g == 0:
                start_rs(c, 0)
            elif g <= 3:
                cur[c].wait_recv()
                rs_add(c, g - 1)
                if g < 3:
                    start_rs(c, g)
                else:
                    start_ag(c, 0)
            else:
                cur[c].wait_recv()
                if g < 6:
                    start_ag(c, g - 3)

        for c in range(_N_CHUNKS):
            r0, nr = _CHUNK_OFFS[c], _CHUNK_ROWS[c]
            out_ref[pl.ds(r0, nr), :] = jnp.dot(
                a_ref[pl.ds(r0, nr), :], b_ref[...],
                preferred_element_type=jnp.float32,
            )
            if c < 6:
                do_step(c, 0)

        for g in range(1, 9):
            if 0 <= g - 2 <= 6:
                for r in range(3):
                    do_step(6 + r, g - 2)
            if 1 <= g - 1 <= 6:
                for r in range(3):
                    do_step(3 + r, g - 1)
            if g <= 6:
                for r in range(3):
                    do_step(r, g)

        for rdma in all_rdmas:
            rdma.wait_send()

        @functools.partial(
            pl.run_scoped, second_barrier=pltpu.SemaphoreType.REGULAR
        )
        def _(second_barrier):
            for pt in partner:
                pl.semaphore_signal(
                    second_barrier, inc=1,
                    device_id=(pt,), device_id_type=pl.DeviceIdType.MESH,
                )
            pl.semaphore_wait(second_barrier, len(partner))

    return pl.pallas_call(
        body,
        out_shape=jax.ShapeDtypeStruct((m, n), jnp.float32),
        in_specs=[
            pl.BlockSpec(memory_space=pltpu.VMEM),
            pl.BlockSpec(memory_space=pltpu.VMEM),
        ],
        out_specs=pl.BlockSpec(memory_space=pltpu.VMEM),
        scratch_shapes=[
            pltpu.VMEM((comm_rows, n), jnp.float32),
            pltpu.SemaphoreType.DMA((_N_CHUNKS, 3)),
            pltpu.SemaphoreType.DMA((_N_CHUNKS, 3)),
            pltpu.SemaphoreType.DMA((_N_CHUNKS, 3)),
            pltpu.SemaphoreType.DMA((_N_CHUNKS, 3)),
        ],
        compiler_params=pltpu.CompilerParams(collective_id=0),
    )(A, B)
